# baseline (speedup 1.0000x reference)
import jax
import jax.numpy as jnp
from jax import lax
from jax.experimental import pallas as pl
from jax.experimental.pallas import tpu as pltpu
from jax.experimental.pallas import tpu_sc as plsc

N = 10000
E = 320000
D = 128
H = D // 2
L = 16
NS = 16
NC = 2
NP = 10240
RPS = NP // NS
U = 128
NU = E // U
UPS = NU // NS
UREM = NU - UPS * NS
SLOTS = 4
GROUPS = UPS // SLOTS
NCH = RPS // U


def _body(xr, rows, cols, lamb, out,
          s_acc, deg_acc,
          rows_all, idx2_all, feats, tailb, nidx, onesb, degb, invb, lvb,
          gsem, ssem, dsem, tsem):
    c = lax.axis_index("c")
    s = lax.axis_index("s")
    cc = c
    cfull = jnp.full((L,), cc, jnp.int32)

    pltpu.sync_copy(lamb, lvb)
    lamv = jnp.maximum(lvb[...], 0.0) + 1.0
    vself = (2.0 * lamv - 2.0) / lamv
    vedge = 2.0 / lamv

    def _zf(r, _):
        for j in range(H // L):
            feats[0, r, pl.ds(j * L, L)] = jnp.zeros((L,), jnp.float32)
        return 0
    lax.fori_loop(0, U, _zf, 0)

    r0 = s * RPS
    for k in range(NCH):
        pltpu.sync_copy(feats.at[0], s_acc.at[pl.ds(r0 + k * U, U)])

    def _zd(i, _):
        degb[0, pl.ds(i * L, L)] = jnp.zeros((L,), jnp.float32)
        return 0
    lax.fori_loop(0, U // L, _zd, 0)
    for k in range(NCH):
        pltpu.sync_copy(degb.at[0], deg_acc.at[pl.ds(r0 + k * U, U)])

    def _ones(i, _):
        onesb[pl.ds(i * L, L)] = jnp.ones((L,), jnp.float32)
        return 0
    lax.fori_loop(0, U // L, _ones, 0)

    plsc.subcore_barrier()

    u0 = s * UPS
    pltpu.sync_copy(rows.at[pl.ds(u0, UPS)], rows_all)
    pltpu.sync_copy(cols.at[pl.ds(u0, UPS)], idx2_all)

    def _ix(u, _):
        for j in range(U // L):
            v = idx2_all[u, pl.ds(j * L, L)]
            idx2_all[u, pl.ds(j * L, L)] = v + v + cfull
        return 0
    lax.fori_loop(0, SLOTS, _ix, 0)

    def _group(g, _):
        for j in range(SLOTS):
            @pl.when(g > 0)
            def _():
                pltpu.make_async_copy(
                    feats.at[j],
                    s_acc.at[rows_all.at[g * SLOTS + j - SLOTS]],
                    ssem.at[j]).wait()
                pltpu.make_async_copy(
                    onesb,
                    deg_acc.at[rows_all.at[g * SLOTS + j - SLOTS]],
                    dsem.at[j]).wait()
            u = g * SLOTS + j
            pltpu.async_copy(xr.at[idx2_all.at[u]], feats.at[j], gsem.at[j])

        @pl.when(g == 0)
        def _():
            lax.fori_loop(SLOTS, UPS, _ix, 0)

        for j in range(SLOTS):
            u = g * SLOTS + j
            pltpu.make_async_copy(
                xr.at[idx2_all.at[u]], feats.at[j], gsem.at[j]).wait()
            pltpu.async_copy(
                feats.at[j], s_acc.at[rows_all.at[u]], ssem.at[j], add=True)
            pltpu.async_copy(
                onesb, deg_acc.at[rows_all.at[u]], dsem.at[j], add=True)
        return 0
    lax.fori_loop(0, GROUPS, _group, 0)

    for j in range(SLOTS):
        u = (GROUPS - 1) * SLOTS + j
        pltpu.make_async_copy(
            feats.at[j], s_acc.at[rows_all.at[u]], ssem.at[j]).wait()
        pltpu.make_async_copy(
            onesb, deg_acc.at[rows_all.at[u]], dsem.at[j]).wait()

    @pl.when(s < UREM)
    def _():
        pltpu.sync_copy(rows.at[UPS * NS + s], tailb.at[0])
        pltpu.sync_copy(cols.at[UPS * NS + s], tailb.at[1])

        def _tix(j, _):
            v = tailb[1, pl.ds(j * L, L)]
            tailb[1, pl.ds(j * L, L)] = v + v + cfull
            return 0
        lax.fori_loop(0, U // L, _tix, 0)
        pltpu.async_copy(xr.at[tailb.at[1]], feats.at[0], tsem).wait()
        pltpu.sync_copy(feats.at[0], s_acc.at[tailb.at[0]], add=True)
        pltpu.sync_copy(onesb, deg_acc.at[tailb.at[0]], add=True)

    plsc.subcore_barrier()

    lane = lax.iota(jnp.int32, L)

    def _fire_loads(k):
        p = k % 2
        rb = r0 + k * U
        pltpu.async_copy(s_acc.at[pl.ds(rb, U)], feats.at[2 * p],
                         gsem.at[2 * p])
        pltpu.async_copy(deg_acc.at[pl.ds(rb, U)], degb.at[p], dsem.at[p])

        def _nix(j, _):
            base2 = 2 * (rb + j * L) + cc
            idx = jnp.full((L,), base2, jnp.int32) + 2 * lane
            nidx[p, pl.ds(j * L, L)] = jnp.minimum(idx, 2 * N - 1)
            return 0
        lax.fori_loop(0, U // L, _nix, 0)
        pltpu.async_copy(xr.at[nidx.at[p]], feats.at[2 * p + 1],
                         gsem.at[2 * p + 1])

    def _wait_loads(k):
        p = k % 2
        rb = r0 + k * U
        pltpu.make_async_copy(s_acc.at[pl.ds(rb, U)], feats.at[2 * p],
                              gsem.at[2 * p]).wait()
        pltpu.make_async_copy(deg_acc.at[pl.ds(rb, U)], degb.at[p],
                              dsem.at[p]).wait()
        pltpu.make_async_copy(xr.at[nidx.at[p]], feats.at[2 * p + 1],
                              gsem.at[2 * p + 1]).wait()

    def _out_ref(k):
        rb = r0 + k * U
        return out.at[pl.ds(rb, U), pl.ds(c * H, H)]

    _fire_loads(0)
    for k in range(NCH):
        p = k % 2
        rb = r0 + k * U
        if k + 1 < NCH:
            if k >= 1:
                @pl.when(r0 + (k - 1) * U + U <= N)
                def _():
                    pltpu.make_async_copy(
                        feats.at[2 * (1 - p)], _out_ref(k - 1),
                        ssem.at[1 - p]).wait()
            _fire_loads(k + 1)
        _wait_loads(k)

        def _inv(i, _):
            d16 = degb[p, pl.ds(i * L, L)]
            invb[pl.ds(i * L, L)] = 1.0 / (vself + vedge * d16)
            return 0
        lax.fori_loop(0, U // L, _inv, 0)

        def _norm(i, _):
            iv16 = invb[pl.ds(i * L, L)]
            for kk in range(L):
                r = i * L + kk
                iv = jnp.full((L,), iv16[kk], jnp.float32)
                for j in range(H // L):
                    s16 = feats[2 * p, r, pl.ds(j * L, L)]
                    x16 = feats[2 * p + 1, r, pl.ds(j * L, L)]
                    feats[2 * p, r, pl.ds(j * L, L)] = (
                        (vself * x16 + vedge * s16) * iv)
            return 0
        lax.fori_loop(0, U // L, _norm, 0)

        @pl.when(rb + U <= N)
        def _():
            pltpu.async_copy(feats.at[2 * p], _out_ref(k), ssem.at[p])

        @pl.when(rb == N - 16)
        def _():
            pltpu.sync_copy(feats.at[2 * p].at[pl.ds(0, 16)],
                            out.at[pl.ds(N - 16, 16), pl.ds(c * H, H)])

    for k in (NCH - 2, NCH - 1):
        p = k % 2

        @pl.when(r0 + k * U + U <= N)
        def _():
            pltpu.make_async_copy(
                feats.at[2 * p], _out_ref(k), ssem.at[p]).wait()


@jax.jit
def _run(xr, rows, cols, lamb):
    mesh = plsc.VectorSubcoreMesh(core_axis_name="c", subcore_axis_name="s")
    kern = pl.kernel(
        _body,
        out_type=jax.ShapeDtypeStruct((N, D), jnp.float32),
        mesh=mesh,
        scratch_types=[
            pltpu.VMEM_SHARED((NP, H), jnp.float32),
            pltpu.VMEM_SHARED((NP,), jnp.float32),
            pltpu.VMEM((UPS, U), jnp.int32),
            pltpu.VMEM((UPS, U), jnp.int32),
            pltpu.VMEM((SLOTS, U, H), jnp.float32),
            pltpu.VMEM((2, U), jnp.int32),
            pltpu.VMEM((2, U), jnp.int32),
            pltpu.VMEM((U,), jnp.float32),
            pltpu.VMEM((2, U), jnp.float32),
            pltpu.VMEM((U,), jnp.float32),
            pltpu.VMEM((L,), jnp.float32),
            pltpu.SemaphoreType.DMA((SLOTS,)),
            pltpu.SemaphoreType.DMA((SLOTS,)),
            pltpu.SemaphoreType.DMA((SLOTS,)),
            pltpu.SemaphoreType.DMA,
        ],
        compiler_params=pltpu.CompilerParams(use_tc_tiling_on_sc=False),
    )
    return kern(xr, rows, cols, lamb)


def kernel(x, edge_index, lambda_param):
    xr = x.reshape(2 * N, H)
    rows = edge_index[0].reshape(NU, U)
    cols = edge_index[1].reshape(NU, U)
    lamb = jnp.full((L,), lambda_param, jnp.float32)
    return _run(xr, rows, cols, lamb)


# speedup vs baseline: 1.0052x; 1.0052x over previous
import jax
import jax.numpy as jnp
from jax import lax
from jax.experimental import pallas as pl
from jax.experimental.pallas import tpu as pltpu
from jax.experimental.pallas import tpu_sc as plsc

N = 10000
E = 320000
D = 128
H = D // 2
L = 16
NS = 16
NC = 2
NP = 10240
RPS = NP // NS
U = 128
NU = E // U
UPS = NU // NS
UREM = NU - UPS * NS
SLOTS = 4
GROUPS = UPS // SLOTS
NCH = RPS // U


def _body(xr, rows, cols, lamb, out,
          s_acc, deg_acc,
          rows_all, idx2_all, feats, tailb, nidx, onesb, degb, invb, lsm,
          gsem, ssem, dsem, tsem):
    c = lax.axis_index("c")
    s = lax.axis_index("s")
    cc = c
    cfull = jnp.full((L,), cc, jnp.int32)

    pltpu.sync_copy(lamb, lsm)
    lamv = jnp.maximum(lsm[...], 0.0) + 1.0
    vself = (2.0 * lamv - 2.0) / lamv
    vedge = 2.0 / lamv

    def _zf(r, _):
        for j in range(H // L):
            feats[0, r, pl.ds(j * L, L)] = jnp.zeros((L,), jnp.float32)
        return 0
    lax.fori_loop(0, U, _zf, 0)

    def _zd(i, _):
        degb[0, pl.ds(i * L, L)] = jnp.zeros((L,), jnp.float32)
        return 0
    lax.fori_loop(0, U // L, _zd, 0)

    r0 = s * RPS
    for k in range(NCH):
        pltpu.async_copy(feats.at[0], s_acc.at[pl.ds(r0 + k * U, U)],
                         gsem.at[0])
        pltpu.async_copy(degb.at[0], deg_acc.at[pl.ds(r0 + k * U, U)],
                         dsem.at[0])
    for k in range(NCH):
        pltpu.make_async_copy(feats.at[0], s_acc.at[pl.ds(r0 + k * U, U)],
                              gsem.at[0]).wait()
        pltpu.make_async_copy(degb.at[0], deg_acc.at[pl.ds(r0 + k * U, U)],
                              dsem.at[0]).wait()

    def _ones(i, _):
        onesb[pl.ds(i * L, L)] = jnp.ones((L,), jnp.float32)
        return 0
    lax.fori_loop(0, U // L, _ones, 0)

    plsc.subcore_barrier()

    u0 = s * UPS
    pltpu.sync_copy(rows.at[pl.ds(u0, UPS)], rows_all)
    pltpu.sync_copy(cols.at[pl.ds(u0, UPS)], idx2_all)

    def _ix(u, _):
        for j in range(U // L):
            v = idx2_all[u, pl.ds(j * L, L)]
            idx2_all[u, pl.ds(j * L, L)] = v + v + cfull
        return 0
    lax.fori_loop(0, SLOTS, _ix, 0)

    def _group(g, _):
        for j in range(SLOTS):
            @pl.when(g > 0)
            def _():
                pltpu.make_async_copy(
                    feats.at[j],
                    s_acc.at[rows_all.at[g * SLOTS + j - SLOTS]],
                    ssem.at[j]).wait()
                pltpu.make_async_copy(
                    onesb,
                    deg_acc.at[rows_all.at[g * SLOTS + j - SLOTS]],
                    dsem.at[j]).wait()
            u = g * SLOTS + j
            pltpu.async_copy(xr.at[idx2_all.at[u]], feats.at[j], gsem.at[j])

        @pl.when(g == 0)
        def _():
            lax.fori_loop(SLOTS, UPS, _ix, 0)

        for j in range(SLOTS):
            u = g * SLOTS + j
            pltpu.make_async_copy(
                xr.at[idx2_all.at[u]], feats.at[j], gsem.at[j]).wait()
            pltpu.async_copy(
                feats.at[j], s_acc.at[rows_all.at[u]], ssem.at[j], add=True)
            pltpu.async_copy(
                onesb, deg_acc.at[rows_all.at[u]], dsem.at[j], add=True)
        return 0
    lax.fori_loop(0, GROUPS, _group, 0)

    for j in range(SLOTS):
        u = (GROUPS - 1) * SLOTS + j
        pltpu.make_async_copy(
            feats.at[j], s_acc.at[rows_all.at[u]], ssem.at[j]).wait()
        pltpu.make_async_copy(
            onesb, deg_acc.at[rows_all.at[u]], dsem.at[j]).wait()

    def _tail_unit(u):
        pltpu.sync_copy(rows.at[u], tailb.at[0])
        pltpu.sync_copy(cols.at[u], tailb.at[1])

        def _tix(j, _):
            v = tailb[1, pl.ds(j * L, L)]
            tailb[1, pl.ds(j * L, L)] = v + v + cfull
            return 0
        lax.fori_loop(0, U // L, _tix, 0)
        pltpu.async_copy(xr.at[tailb.at[1]], feats.at[0], tsem).wait()
        pltpu.sync_copy(feats.at[0], s_acc.at[tailb.at[0]], add=True)
        pltpu.sync_copy(onesb, deg_acc.at[tailb.at[0]], add=True)

    for u_left in range(GROUPS * SLOTS, UPS):
        _tail_unit(u0 + u_left)

    @pl.when(s < UREM)
    def _():
        _tail_unit(UPS * NS + s)

    plsc.subcore_barrier()

    lane = lax.iota(jnp.int32, L)

    def _fire_loads(k):
        p = k % 2
        rb = r0 + k * U
        pltpu.async_copy(s_acc.at[pl.ds(rb, U)], feats.at[2 * p],
                         gsem.at[2 * p])
        pltpu.async_copy(deg_acc.at[pl.ds(rb, U)], degb.at[p], dsem.at[p])

        def _nix(j, _):
            base2 = 2 * (rb + j * L) + cc
            idx = jnp.full((L,), base2, jnp.int32) + 2 * lane
            nidx[p, pl.ds(j * L, L)] = jnp.minimum(idx, 2 * N - 1)
            return 0
        lax.fori_loop(0, U // L, _nix, 0)
        pltpu.async_copy(xr.at[nidx.at[p]], feats.at[2 * p + 1],
                         gsem.at[2 * p + 1])

    def _wait_loads(k):
        p = k % 2
        rb = r0 + k * U
        pltpu.make_async_copy(s_acc.at[pl.ds(rb, U)], feats.at[2 * p],
                              gsem.at[2 * p]).wait()
        pltpu.make_async_copy(deg_acc.at[pl.ds(rb, U)], degb.at[p],
                              dsem.at[p]).wait()
        pltpu.make_async_copy(xr.at[nidx.at[p]], feats.at[2 * p + 1],
                              gsem.at[2 * p + 1]).wait()

    def _out_ref(k):
        rb = r0 + k * U
        return out.at[pl.ds(rb, U), pl.ds(c * H, H)]

    _fire_loads(0)
    for k in range(NCH):
        p = k % 2
        rb = r0 + k * U
        if k + 1 < NCH:
            if k >= 1:
                @pl.when(r0 + (k - 1) * U + U <= N)
                def _():
                    pltpu.make_async_copy(
                        feats.at[2 * (1 - p)], _out_ref(k - 1),
                        ssem.at[1 - p]).wait()
            _fire_loads(k + 1)
        _wait_loads(k)

        def _inv(i, _):
            d16 = degb[p, pl.ds(i * L, L)]
            invb[pl.ds(i * L, L)] = 1.0 / (vself + vedge * d16)
            return 0
        lax.fori_loop(0, U // L, _inv, 0)

        def _norm(i, _):
            iv16 = invb[pl.ds(i * L, L)]
            for kk in range(L):
                r = i * L + kk
                iv = jnp.full((L,), iv16[kk], jnp.float32)
                for j in range(H // L):
                    s16 = feats[2 * p, r, pl.ds(j * L, L)]
                    x16 = feats[2 * p + 1, r, pl.ds(j * L, L)]
                    feats[2 * p, r, pl.ds(j * L, L)] = (
                        (vself * x16 + vedge * s16) * iv)
            return 0
        lax.fori_loop(0, U // L, _norm, 0)

        @pl.when(rb + U <= N)
        def _():
            pltpu.async_copy(feats.at[2 * p], _out_ref(k), ssem.at[p])

        @pl.when(rb == N - 16)
        def _():
            pltpu.sync_copy(feats.at[2 * p].at[pl.ds(0, 16)],
                            out.at[pl.ds(N - 16, 16), pl.ds(c * H, H)])

    for k in (NCH - 2, NCH - 1):
        p = k % 2

        @pl.when(r0 + k * U + U <= N)
        def _():
            pltpu.make_async_copy(
                feats.at[2 * p], _out_ref(k), ssem.at[p]).wait()


@jax.jit
def _run(xr, rows, cols, lamb):
    mesh = plsc.VectorSubcoreMesh(core_axis_name="c", subcore_axis_name="s")
    kern = pl.kernel(
        _body,
        out_type=jax.ShapeDtypeStruct((N, D), jnp.float32),
        mesh=mesh,
        scratch_types=[
            pltpu.VMEM_SHARED((NP, H), jnp.float32),
            pltpu.VMEM_SHARED((NP,), jnp.float32),
            pltpu.VMEM((UPS, U), jnp.int32),
            pltpu.VMEM((UPS, U), jnp.int32),
            pltpu.VMEM((SLOTS, U, H), jnp.float32),
            pltpu.VMEM((2, U), jnp.int32),
            pltpu.VMEM((2, U), jnp.int32),
            pltpu.VMEM((U,), jnp.float32),
            pltpu.VMEM((2, U), jnp.float32),
            pltpu.VMEM((U,), jnp.float32),
            pltpu.VMEM((L,), jnp.float32),
            pltpu.SemaphoreType.DMA((SLOTS,)),
            pltpu.SemaphoreType.DMA((SLOTS,)),
            pltpu.SemaphoreType.DMA((SLOTS,)),
            pltpu.SemaphoreType.DMA,
        ],
        compiler_params=pltpu.CompilerParams(use_tc_tiling_on_sc=False),
    )
    return kern(xr, rows, cols, lamb)


def kernel(x, edge_index, lambda_param):
    xr = x.reshape(2 * N, H)
    rows = edge_index[0].reshape(NU, U)
    cols = edge_index[1].reshape(NU, U)
    lamb = jnp.full((L,), lambda_param, jnp.float32)
    return _run(xr, rows, cols, lamb)


# speedup vs baseline: 1.0068x; 1.0015x over previous
import jax
import jax.numpy as jnp
from jax import lax
from jax.experimental import pallas as pl
from jax.experimental.pallas import tpu as pltpu
from jax.experimental.pallas import tpu_sc as plsc

N = 10000
E = 320000
D = 128
H = D // 2
L = 16
NS = 16
NC = 2
NP = 10240
RPS = NP // NS
U = 128
NU = E // U
UPS = NU // NS
UREM = NU - UPS * NS
SLOTS = 4
GROUPS = UPS // SLOTS
NCH = RPS // U


def _body(xr, rows, cols, lamb, out,
          s_acc, deg_acc,
          rows_all, idx2_all, feats, tailb, nidx, deghist, identb,
          degb, invb, lsm,
          gsem, ssem, dsem, tsem):
    c = lax.axis_index("c")
    s = lax.axis_index("s")
    cc = c
    cfull = jnp.full((L,), cc, jnp.int32)

    pltpu.sync_copy(lamb, lsm)
    lamv = jnp.maximum(lsm[...], 0.0) + 1.0
    vself = (2.0 * lamv - 2.0) / lamv
    vedge = 2.0 / lamv

    def _zf(r, _):
        for j in range(H // L):
            feats[0, r, pl.ds(j * L, L)] = jnp.zeros((L,), jnp.float32)
        return 0
    lax.fori_loop(0, U, _zf, 0)

    def _zh(r, _):
        for j in range(U // L):
            deghist[r, pl.ds(j * L, L)] = jnp.zeros((L,), jnp.float32)
        return 0
    lax.fori_loop(0, NP // U, _zh, 0)
    for i in range(NP // U // L):
        identb[0, pl.ds(i * L, L)] = (
            jnp.full((L,), i * L, jnp.int32) + lax.iota(jnp.int32, L))

    def _zd(i, _):
        degb[0, pl.ds(i * L, L)] = jnp.zeros((L,), jnp.float32)
        degb[1, pl.ds(i * L, L)] = jnp.zeros((L,), jnp.float32)
        return 0
    lax.fori_loop(0, U // L, _zd, 0)

    r0 = s * RPS
    dr0 = s * (RPS // U)
    for k in range(NCH):
        pltpu.async_copy(feats.at[0], s_acc.at[pl.ds(r0 + k * U, U)],
                         gsem.at[0])
        pltpu.async_copy(degb.at[0], deg_acc.at[dr0 + k], dsem.at[0])
    for k in range(NCH):
        pltpu.make_async_copy(feats.at[0], s_acc.at[pl.ds(r0 + k * U, U)],
                              gsem.at[0]).wait()
        pltpu.make_async_copy(degb.at[0], deg_acc.at[dr0 + k],
                              dsem.at[0]).wait()

    plsc.subcore_barrier()
    ones16 = jnp.ones((L,), jnp.float32)

    u0 = s * UPS
    pltpu.sync_copy(rows.at[pl.ds(u0, UPS)], rows_all)
    pltpu.sync_copy(cols.at[pl.ds(u0, UPS)], idx2_all)

    def _ix(u, _):
        for j in range(U // L):
            v = idx2_all[u, pl.ds(j * L, L)]
            idx2_all[u, pl.ds(j * L, L)] = v + v + cfull
        return 0
    lax.fori_loop(0, SLOTS, _ix, 0)

    def _hist(idxv):
        ri = lax.shift_right_logical(idxv, 7)
        ci = jnp.bitwise_and(idxv, jnp.full((L,), U - 1, jnp.int32))
        plsc.addupdate_scatter(deghist, [ri, ci], ones16)

    def _group(g, _):
        for j in range(SLOTS):
            @pl.when(g > 0)
            def _():
                pltpu.make_async_copy(
                    feats.at[j],
                    s_acc.at[rows_all.at[g * SLOTS + j - SLOTS]],
                    ssem.at[j]).wait()
            u = g * SLOTS + j
            pltpu.async_copy(xr.at[idx2_all.at[u]], feats.at[j], gsem.at[j])

        @pl.when(g == 0)
        def _():
            lax.fori_loop(SLOTS, UPS, _ix, 0)

        for j in range(SLOTS):
            u = g * SLOTS + j
            for jj in range(U // L):
                _hist(rows_all[u, pl.ds(jj * L, L)])
            pltpu.make_async_copy(
                xr.at[idx2_all.at[u]], feats.at[j], gsem.at[j]).wait()
            pltpu.async_copy(
                feats.at[j], s_acc.at[rows_all.at[u]], ssem.at[j], add=True)
        return 0
    lax.fori_loop(0, GROUPS, _group, 0)

    for j in range(SLOTS):
        u = (GROUPS - 1) * SLOTS + j
        pltpu.make_async_copy(
            feats.at[j], s_acc.at[rows_all.at[u]], ssem.at[j]).wait()

    def _tail_unit(u):
        pltpu.sync_copy(rows.at[u], tailb.at[0])
        pltpu.sync_copy(cols.at[u], tailb.at[1])

        def _tix(j, _):
            v = tailb[1, pl.ds(j * L, L)]
            tailb[1, pl.ds(j * L, L)] = v + v + cfull
            return 0
        lax.fori_loop(0, U // L, _tix, 0)
        for jj in range(U // L):
            _hist(tailb[0, pl.ds(jj * L, L)])
        pltpu.async_copy(xr.at[tailb.at[1]], feats.at[0], tsem).wait()
        pltpu.sync_copy(feats.at[0], s_acc.at[tailb.at[0]], add=True)

    for u_left in range(GROUPS * SLOTS, UPS):
        _tail_unit(u0 + u_left)

    @pl.when(s < UREM)
    def _():
        _tail_unit(UPS * NS + s)

    pltpu.sync_copy(deghist, deg_acc.at[identb.at[0]], add=True)

    plsc.subcore_barrier()

    lane = lax.iota(jnp.int32, L)

    def _fire_loads(k):
        p = k % 2
        rb = r0 + k * U
        pltpu.async_copy(s_acc.at[pl.ds(rb, U)], feats.at[2 * p],
                         gsem.at[2 * p])
        pltpu.async_copy(deg_acc.at[dr0 + k], degb.at[p], dsem.at[p])

        def _nix(j, _):
            base2 = 2 * (rb + j * L) + cc
            idx = jnp.full((L,), base2, jnp.int32) + 2 * lane
            nidx[p, pl.ds(j * L, L)] = jnp.minimum(idx, 2 * N - 1)
            return 0
        lax.fori_loop(0, U // L, _nix, 0)
        pltpu.async_copy(xr.at[nidx.at[p]], feats.at[2 * p + 1],
                         gsem.at[2 * p + 1])

    def _wait_loads(k):
        p = k % 2
        rb = r0 + k * U
        pltpu.make_async_copy(s_acc.at[pl.ds(rb, U)], feats.at[2 * p],
                              gsem.at[2 * p]).wait()
        pltpu.make_async_copy(deg_acc.at[dr0 + k], degb.at[p],
                              dsem.at[p]).wait()
        pltpu.make_async_copy(xr.at[nidx.at[p]], feats.at[2 * p + 1],
                              gsem.at[2 * p + 1]).wait()

    def _out_ref(k):
        rb = r0 + k * U
        return out.at[pl.ds(rb, U), pl.ds(c * H, H)]

    _fire_loads(0)
    for k in range(NCH):
        p = k % 2
        rb = r0 + k * U
        if k + 1 < NCH:
            if k >= 1:
                @pl.when(r0 + (k - 1) * U + U <= N)
                def _():
                    pltpu.make_async_copy(
                        feats.at[2 * (1 - p)], _out_ref(k - 1),
                        ssem.at[1 - p]).wait()
            _fire_loads(k + 1)
        _wait_loads(k)

        def _inv(i, _):
            d16 = degb[p, pl.ds(i * L, L)]
            invb[pl.ds(i * L, L)] = 1.0 / (vself + vedge * d16)
            return 0
        lax.fori_loop(0, U // L, _inv, 0)

        def _norm(i, _):
            iv16 = invb[pl.ds(i * L, L)]
            for kk in range(L):
                r = i * L + kk
                iv = jnp.full((L,), iv16[kk], jnp.float32)
                for j in range(H // L):
                    s16 = feats[2 * p, r, pl.ds(j * L, L)]
                    x16 = feats[2 * p + 1, r, pl.ds(j * L, L)]
                    feats[2 * p, r, pl.ds(j * L, L)] = (
                        (vself * x16 + vedge * s16) * iv)
            return 0
        lax.fori_loop(0, U // L, _norm, 0)

        @pl.when(rb + U <= N)
        def _():
            pltpu.async_copy(feats.at[2 * p], _out_ref(k), ssem.at[p])

        @pl.when(rb == N - 16)
        def _():
            pltpu.sync_copy(feats.at[2 * p].at[pl.ds(0, 16)],
                            out.at[pl.ds(N - 16, 16), pl.ds(c * H, H)])

    for k in (NCH - 2, NCH - 1):
        p = k % 2

        @pl.when(r0 + k * U + U <= N)
        def _():
            pltpu.make_async_copy(
                feats.at[2 * p], _out_ref(k), ssem.at[p]).wait()


@jax.jit
def _run(xr, rows, cols, lamb):
    mesh = plsc.VectorSubcoreMesh(core_axis_name="c", subcore_axis_name="s")
    kern = pl.kernel(
        _body,
        out_type=jax.ShapeDtypeStruct((N, D), jnp.float32),
        mesh=mesh,
        scratch_types=[
            pltpu.VMEM_SHARED((NP, H), jnp.float32),
            pltpu.VMEM_SHARED((NP // U, U), jnp.float32),
            pltpu.VMEM((UPS, U), jnp.int32),
            pltpu.VMEM((UPS, U), jnp.int32),
            pltpu.VMEM((SLOTS, U, H), jnp.float32),
            pltpu.VMEM((2, U), jnp.int32),
            pltpu.VMEM((2, U), jnp.int32),
            pltpu.VMEM((NP // U, U), jnp.float32),
            pltpu.VMEM((1, NP // U), jnp.int32),
            pltpu.VMEM((2, U), jnp.float32),
            pltpu.VMEM((U,), jnp.float32),
            pltpu.VMEM((L,), jnp.float32),
            pltpu.SemaphoreType.DMA((SLOTS,)),
            pltpu.SemaphoreType.DMA((SLOTS,)),
            pltpu.SemaphoreType.DMA((SLOTS,)),
            pltpu.SemaphoreType.DMA,
        ],
        compiler_params=pltpu.CompilerParams(
            use_tc_tiling_on_sc=False, needs_layout_passes=False),
    )
    return kern(xr, rows, cols, lamb)


def kernel(x, edge_index, lambda_param):
    xr = x.reshape(2 * N, H)
    rows = edge_index[0].reshape(NU, U)
    cols = edge_index[1].reshape(NU, U)
    lamb = jnp.full((L,), lambda_param, jnp.float32)
    return _run(xr, rows, cols, lamb)
